# Initial kernel scaffold; baseline (speedup 1.0000x reference)
import functools

import jax
import jax.numpy as jnp
from jax import lax
from jax.experimental import pallas as pl
from jax.experimental.pallas import tpu as pltpu
from jax.experimental.pallas import tpu_sc as plsc

N = 10000
E = 320000
H = 128
NI = 16
D = 2 * H + NI

NC = 2
NS = 16
NW = NC * NS
L = 16

EPW = E // NW
CH = 80
NCHUNK = EPW // CH

FPW = 2 * E // NW
HCH = 80
NHCHUNK = FPW // HCH

_mesh = plsc.VectorSubcoreMesh(core_axis_name="c", subcore_axis_name="s")


@functools.partial(
    pl.kernel,
    out_type=jax.ShapeDtypeStruct((NC, 2 * N), jnp.float32),
    mesh=_mesh,
    scratch_types=[
        pltpu.VMEM((HCH,), jnp.int32),
        pltpu.VMEM((HCH,), jnp.int32),
        pltpu.VMEM((HCH,), jnp.float32),
        pltpu.VMEM_SHARED((2 * N,), jnp.float32),
    ],
)
def _hist_kernel(idx_hbm, zeros_hbm, out_hbm, idx_v, idxa_v, ones_v, acc_sh):
    cid = lax.axis_index("c")
    sid = lax.axis_index("s")
    wid = cid * NS + sid

    @pl.when(sid == 0)
    def _():
        pltpu.sync_copy(zeros_hbm, acc_sh)

    for k in range(HCH // L):
        ones_v[pl.ds(k * L, L)] = jnp.full((L,), 1.0, jnp.float32)
    plsc.subcore_barrier()

    base = wid * FPW
    off = jnp.where(wid >= NW // 2, N, 0).astype(jnp.int32)

    def chunk_body(i, carry):
        pltpu.sync_copy(idx_hbm.at[pl.ds(base + i * HCH, HCH)], idx_v)
        for k in range(HCH // L):
            sl = pl.ds(k * L, L)
            idxa_v[sl] = idx_v[sl] + off
        pltpu.sync_copy(ones_v, acc_sh.at[idxa_v], add=True)
        return carry

    lax.fori_loop(0, NHCHUNK, chunk_body, 0)
    plsc.subcore_barrier()

    @pl.when(sid == 0)
    def _():
        pltpu.sync_copy(acc_sh, out_hbm.at[cid])


def _stats_body(cnt_ref, xs_ref, xr_ref, ear_ref, g_ref, b_ref, w1_ref, b1_ref,
                ps_ref, pr_ref, se_ref, b1e_ref):
    c = cnt_ref[0] + cnt_ref[1]
    cs = c[0:1, :]
    cr = c[1:2, :]
    xs = xs_ref[...]
    xr = xr_ref[...]
    inv_e = 1.0 / E

    sum_s = jnp.dot(cs, xs, preferred_element_type=jnp.float32)
    sumsq_s = jnp.dot(cs, xs * xs, preferred_element_type=jnp.float32)
    sum_r = jnp.dot(cr, xr, preferred_element_type=jnp.float32)
    sumsq_r = jnp.dot(cr, xr * xr, preferred_element_type=jnp.float32)
    mean_s = sum_s * inv_e
    var_s = sumsq_s * inv_e - mean_s * mean_s
    mean_r = sum_r * inv_e
    var_r = sumsq_r * inv_e - mean_r * mean_r

    ear = ear_ref[...]
    se_cols = jnp.sum(ear, axis=0)
    sq_cols = jnp.sum(ear * ear, axis=0)
    mean_e = jnp.sum(se_cols.reshape(H // NI, NI), axis=0) * inv_e
    msq_e = jnp.sum(sq_cols.reshape(H // NI, NI), axis=0) * inv_e
    var_e = msq_e - mean_e * mean_e

    mean = jnp.concatenate([mean_s[0], mean_r[0], mean_e]).reshape(1, D)
    var = jnp.concatenate([var_s[0], var_r[0], var_e]).reshape(1, D)
    scale = g_ref[...] * jax.lax.rsqrt(var + 1e-5)
    shift = b_ref[...] - mean * scale

    b1e_ref[...] = b1_ref[...] + jnp.dot(
        shift, w1_ref[...], preferred_element_type=jnp.float32)
    ps_ref[...] = jnp.dot(xs * scale[:, :H], w1_ref[:H, :],
                          preferred_element_type=jnp.float32)
    pr_ref[...] = jnp.dot(xr * scale[:, H:2 * H], w1_ref[H:2 * H, :],
                          preferred_element_type=jnp.float32)
    se_ref[...] = scale[:, 2 * H:]


def _stats_call(counts, x_send, x_rec, ea_r, gamma2, beta2, W1, b12):
    return pl.pallas_call(
        _stats_body,
        out_shape=[
            jax.ShapeDtypeStruct((N, H), jnp.float32),
            jax.ShapeDtypeStruct((N, H), jnp.float32),
            jax.ShapeDtypeStruct((1, NI), jnp.float32),
            jax.ShapeDtypeStruct((1, H), jnp.float32),
        ],
    )(counts, x_send, x_rec, ea_r, gamma2, beta2, W1, b12)


_EB = 8000

def _pe_body(ea_ref, se_ref, w1e_ref, b1e_ref, pe_ref):
    pe_ref[...] = jnp.dot(ea_ref[...] * se_ref[...], w1e_ref[...],
                          preferred_element_type=jnp.float32) + b1e_ref[...]


def _pe_call(edge_attr, se, w1e, b1e):
    grid = (E // _EB,)
    return pl.pallas_call(
        _pe_body,
        grid=grid,
        in_specs=[
            pl.BlockSpec((_EB, NI), lambda i: (i, 0)),
            pl.BlockSpec((1, NI), lambda i: (0, 0)),
            pl.BlockSpec((NI, H), lambda i: (0, 0)),
            pl.BlockSpec((1, H), lambda i: (0, 0)),
        ],
        out_specs=pl.BlockSpec((_EB, H), lambda i: (i, 0)),
        out_shape=jax.ShapeDtypeStruct((E, H), jnp.float32),
    )(edge_attr, se, w1e, b1e)


@functools.partial(
    pl.kernel,
    out_type=jax.ShapeDtypeStruct((NC, N, H), jnp.float32),
    mesh=_mesh,
    scratch_types=[
        pltpu.VMEM((CH,), jnp.int32),
        pltpu.VMEM((CH,), jnp.int32),
        pltpu.VMEM((CH, H), jnp.float32),
        pltpu.VMEM((CH, H), jnp.float32),
        pltpu.VMEM((CH, H), jnp.float32),
        pltpu.VMEM((CH, H), jnp.float32),
        pltpu.VMEM((H,), jnp.float32),
        pltpu.VMEM((L,), jnp.float32),
        pltpu.VMEM_SHARED((N, H), jnp.float32),
        pltpu.SemaphoreType.DMA,
        pltpu.SemaphoreType.DMA,
        pltpu.SemaphoreType.DMA,
    ],
)
def _main_kernel(ps_hbm, pr_hbm, pe_hbm, is_hbm, ir_hbm, w2_hbm, b2_hbm,
                 zeros_hbm, out_hbm, is_v, ir_v, rs_v, rr_v, rpe_v, ro_v,
                 w2_v, b2_v, acc_sh, sem1, sem2, sem3):
    cid = lax.axis_index("c")
    sid = lax.axis_index("s")
    wid = cid * NS + sid

    @pl.when(sid == 0)
    def _():
        pltpu.sync_copy(zeros_hbm, acc_sh)

    pltpu.sync_copy(w2_hbm, w2_v)
    pltpu.sync_copy(b2_hbm, b2_v)
    plsc.subcore_barrier()

    base = wid * EPW
    b2v = b2_v[pl.ds(0, L)]

    def chunk_body(i, carry):
        eb = base + i * CH
        pltpu.sync_copy(is_hbm.at[pl.ds(eb, CH)], is_v)
        pltpu.sync_copy(ir_hbm.at[pl.ds(eb, CH)], ir_v)
        cp1 = pltpu.async_copy(ps_hbm.at[is_v], rs_v, sem1)
        cp2 = pltpu.async_copy(pr_hbm.at[ir_v], rr_v, sem2)
        cp3 = pltpu.async_copy(pe_hbm.at[pl.ds(eb, CH)], rpe_v, sem3)
        cp1.wait()
        cp2.wait()
        cp3.wait()

        def edge_body(e, ecarry):
            acc = jnp.zeros((L,), jnp.float32)
            ms = []
            for j in range(H // L):
                sl = pl.ds(j * L, L)
                z = rs_v[e, sl] + rr_v[e, sl] + rpe_v[e, sl]
                m = z / (1.0 + jnp.exp(-z))
                ms.append(m)
                acc = acc + m * w2_v[sl]
            tot = jnp.sum(acc)
            tv = jax.lax.broadcast(tot, (L,)) + b2v
            w = 1.0 / (1.0 + jnp.exp(-tv))
            for j in range(H // L):
                ro_v[e, pl.ds(j * L, L)] = ms[j] * w
            return ecarry

        lax.fori_loop(0, CH, edge_body, 0)
        pltpu.sync_copy(ro_v, acc_sh.at[ir_v], add=True)
        return carry

    lax.fori_loop(0, NCHUNK, chunk_body, 0)
    plsc.subcore_barrier()

    rows = N // NS
    pltpu.sync_copy(acc_sh.at[pl.ds(sid * rows, rows)],
                    out_hbm.at[cid, pl.ds(sid * rows, rows)])


_NB = 2000

def _combine_body(p_ref, o_ref):
    o_ref[...] = p_ref[0] + p_ref[1]


def _combine_call(parts):
    return pl.pallas_call(
        _combine_body,
        grid=(N // _NB,),
        in_specs=[pl.BlockSpec((NC, _NB, H), lambda i: (0, i, 0))],
        out_specs=pl.BlockSpec((_NB, H), lambda i: (i, 0)),
        out_shape=jax.ShapeDtypeStruct((N, H), jnp.float32),
    )(parts)


@jax.jit
def kernel(x_send, x_rec, index, edge_attr, gamma, beta, W1, b1, W2, b2):
    idx_flat = index.reshape(2 * E)
    counts2 = _hist_kernel(idx_flat, jnp.zeros((2 * N,), jnp.float32))
    counts = counts2.reshape(NC, 2, N)

    ea_r = edge_attr.reshape(E * NI // H, H)
    p_send, p_rec, se, b1e = _stats_call(
        counts, x_send, x_rec, ea_r, gamma.reshape(1, D), beta.reshape(1, D),
        W1, b1.reshape(1, H))

    pe = _pe_call(edge_attr, se, W1[2 * H:, :], b1e)

    b2v = jnp.broadcast_to(b2.reshape(1), (L,)).astype(jnp.float32)
    parts = _main_kernel(p_send, p_rec, pe, index[0], index[1], W2[:, 0], b2v,
                         jnp.zeros((N, H), jnp.float32))
    return _combine_call(parts)


# speedup vs baseline: 2.9169x; 2.9169x over previous
import functools

import jax
import jax.numpy as jnp
from jax import lax
from jax.experimental import pallas as pl
from jax.experimental.pallas import tpu as pltpu
from jax.experimental.pallas import tpu_sc as plsc

N = 10000
E = 320000
H = 128
NI = 16
D = 2 * H + NI

NC = 2
NS = 16
NW = NC * NS
L = 16

EPW = E // NW
CH = 80
NCHUNK = EPW // CH

FPW = 2 * E // NW
HCH = 80
NHCHUNK = FPW // HCH

_mesh = plsc.VectorSubcoreMesh(core_axis_name="c", subcore_axis_name="s")


@functools.partial(
    pl.kernel,
    out_type=jax.ShapeDtypeStruct((NC, 2 * N), jnp.float32),
    mesh=_mesh,
    scratch_types=[
        pltpu.VMEM((HCH,), jnp.int32),
        pltpu.VMEM((HCH,), jnp.int32),
        pltpu.VMEM((HCH,), jnp.float32),
        pltpu.VMEM_SHARED((2 * N,), jnp.float32),
    ],
    compiler_params=pltpu.CompilerParams(needs_layout_passes=False),
)
def _hist_kernel(idx_hbm, zeros_hbm, out_hbm, idx_v, idxa_v, ones_v, acc_sh):
    cid = lax.axis_index("c")
    sid = lax.axis_index("s")
    wid = cid * NS + sid

    @pl.when(sid == 0)
    def _():
        pltpu.sync_copy(zeros_hbm, acc_sh)

    for k in range(HCH // L):
        ones_v[pl.ds(k * L, L)] = jnp.full((L,), 1.0, jnp.float32)
    plsc.subcore_barrier()

    base = wid * FPW
    off = jnp.where(wid >= NW // 2, N, 0).astype(jnp.int32)

    def chunk_body(i, carry):
        pltpu.sync_copy(idx_hbm.at[pl.ds(base + i * HCH, HCH)], idx_v)
        for k in range(HCH // L):
            sl = pl.ds(k * L, L)
            idxa_v[sl] = idx_v[sl] + off
        pltpu.sync_copy(ones_v, acc_sh.at[idxa_v], add=True)
        return carry

    lax.fori_loop(0, NHCHUNK, chunk_body, 0)
    plsc.subcore_barrier()

    @pl.when(sid == 0)
    def _():
        pltpu.sync_copy(acc_sh, out_hbm.at[cid])


def _stats_body(cnt_ref, xs_ref, xr_ref, ear_ref, g_ref, b_ref, w1_ref, b1_ref,
                ps_ref, pr_ref, se_ref, b1e_ref):
    c = cnt_ref[0] + cnt_ref[1]
    cs = c[0:1, :]
    cr = c[1:2, :]
    xs = xs_ref[...]
    xr = xr_ref[...]
    inv_e = 1.0 / E

    sum_s = jnp.dot(cs, xs, preferred_element_type=jnp.float32)
    sumsq_s = jnp.dot(cs, xs * xs, preferred_element_type=jnp.float32)
    sum_r = jnp.dot(cr, xr, preferred_element_type=jnp.float32)
    sumsq_r = jnp.dot(cr, xr * xr, preferred_element_type=jnp.float32)
    mean_s = sum_s * inv_e
    var_s = sumsq_s * inv_e - mean_s * mean_s
    mean_r = sum_r * inv_e
    var_r = sumsq_r * inv_e - mean_r * mean_r

    ear = ear_ref[...]
    se_cols = jnp.sum(ear, axis=0, keepdims=True)
    sq_cols = jnp.sum(ear * ear, axis=0, keepdims=True)
    sum_e = se_cols[:, 0:NI]
    sumsq_e = sq_cols[:, 0:NI]
    for g in range(1, H // NI):
        sum_e = sum_e + se_cols[:, g * NI:(g + 1) * NI]
        sumsq_e = sumsq_e + sq_cols[:, g * NI:(g + 1) * NI]
    mean_e = sum_e * inv_e
    msq_e = sumsq_e * inv_e
    var_e = msq_e - mean_e * mean_e

    mean = jnp.concatenate([mean_s, mean_r, mean_e], axis=1)
    var = jnp.concatenate([var_s, var_r, var_e], axis=1)
    scale = g_ref[...] * jax.lax.rsqrt(var + 1e-5)
    shift = b_ref[...] - mean * scale

    b1e_ref[...] = b1_ref[...] + jnp.dot(
        shift, w1_ref[...], preferred_element_type=jnp.float32)
    ps_ref[...] = jnp.dot(xs * scale[:, :H], w1_ref[:H, :],
                          preferred_element_type=jnp.float32)
    pr_ref[...] = jnp.dot(xr * scale[:, H:2 * H], w1_ref[H:2 * H, :],
                          preferred_element_type=jnp.float32)
    se_ref[...] = scale[:, 2 * H:]


def _stats_call(counts, x_send, x_rec, ea_r, gamma2, beta2, W1, b12):
    return pl.pallas_call(
        _stats_body,
        out_shape=[
            jax.ShapeDtypeStruct((N, H), jnp.float32),
            jax.ShapeDtypeStruct((N, H), jnp.float32),
            jax.ShapeDtypeStruct((1, NI), jnp.float32),
            jax.ShapeDtypeStruct((1, H), jnp.float32),
        ],
    )(counts, x_send, x_rec, ea_r, gamma2, beta2, W1, b12)


_EB = 8000

def _pe_body(ea_ref, se_ref, w1e_ref, b1e_ref, pe_ref):
    pe_ref[...] = jnp.dot(ea_ref[...] * se_ref[...], w1e_ref[...],
                          preferred_element_type=jnp.float32) + b1e_ref[...]


def _pe_call(edge_attr, se, w1e, b1e):
    grid = (E // _EB,)
    return pl.pallas_call(
        _pe_body,
        grid=grid,
        in_specs=[
            pl.BlockSpec((_EB, NI), lambda i: (i, 0)),
            pl.BlockSpec((1, NI), lambda i: (0, 0)),
            pl.BlockSpec((NI, H), lambda i: (0, 0)),
            pl.BlockSpec((1, H), lambda i: (0, 0)),
        ],
        out_specs=pl.BlockSpec((_EB, H), lambda i: (i, 0)),
        out_shape=jax.ShapeDtypeStruct((E, H), jnp.float32),
    )(edge_attr, se, w1e, b1e)


@functools.partial(
    pl.kernel,
    out_type=jax.ShapeDtypeStruct((NC, N, H), jnp.float32),
    mesh=_mesh,
    scratch_types=[
        pltpu.VMEM((CH,), jnp.int32),
        pltpu.VMEM((CH,), jnp.int32),
        pltpu.VMEM((CH, H), jnp.float32),
        pltpu.VMEM((CH, H), jnp.float32),
        pltpu.VMEM((CH, H), jnp.float32),
        pltpu.VMEM((CH, H), jnp.float32),
        pltpu.VMEM((H,), jnp.float32),
        pltpu.VMEM((L,), jnp.float32),
        pltpu.VMEM_SHARED((N, H), jnp.float32),
        pltpu.SemaphoreType.DMA,
        pltpu.SemaphoreType.DMA,
        pltpu.SemaphoreType.DMA,
    ],
    compiler_params=pltpu.CompilerParams(needs_layout_passes=False),
)
def _main_kernel(ps_hbm, pr_hbm, pe_hbm, is_hbm, ir_hbm, w2_hbm, b2_hbm,
                 zeros_hbm, out_hbm, is_v, ir_v, rs_v, rr_v, rpe_v, ro_v,
                 w2_v, b2_v, acc_sh, sem1, sem2, sem3):
    cid = lax.axis_index("c")
    sid = lax.axis_index("s")
    wid = cid * NS + sid

    @pl.when(sid == 0)
    def _():
        pltpu.sync_copy(zeros_hbm, acc_sh)

    pltpu.sync_copy(w2_hbm, w2_v)
    pltpu.sync_copy(b2_hbm, b2_v)
    plsc.subcore_barrier()

    base = wid * EPW
    b2v = b2_v[pl.ds(0, L)]

    def chunk_body(i, carry):
        eb = base + i * CH
        pltpu.sync_copy(is_hbm.at[pl.ds(eb, CH)], is_v)
        pltpu.sync_copy(ir_hbm.at[pl.ds(eb, CH)], ir_v)
        cp1 = pltpu.async_copy(ps_hbm.at[is_v], rs_v, sem1)
        cp2 = pltpu.async_copy(pr_hbm.at[ir_v], rr_v, sem2)
        cp3 = pltpu.async_copy(pe_hbm.at[pl.ds(eb, CH)], rpe_v, sem3)
        cp1.wait()
        cp2.wait()
        cp3.wait()

        def edge_body(e, ecarry):
            acc = jnp.zeros((L,), jnp.float32)
            ms = []
            for j in range(H // L):
                sl = pl.ds(j * L, L)
                z = rs_v[e, sl] + rr_v[e, sl] + rpe_v[e, sl]
                m = z / (1.0 + jnp.exp(-z))
                ms.append(m)
                acc = acc + m * w2_v[sl]
            tot = jnp.sum(acc)
            tv = jax.lax.broadcast(tot, (L,)) + b2v
            w = 1.0 / (1.0 + jnp.exp(-tv))
            for j in range(H // L):
                ro_v[e, pl.ds(j * L, L)] = ms[j] * w
            return ecarry

        lax.fori_loop(0, CH, edge_body, 0)
        pltpu.sync_copy(ro_v, acc_sh.at[ir_v], add=True)
        return carry

    lax.fori_loop(0, NCHUNK, chunk_body, 0)
    plsc.subcore_barrier()

    @pl.when(sid < NS - 1)
    def _():
        pltpu.sync_copy(acc_sh.at[pl.ds(sid * 640, 640)],
                        out_hbm.at[cid, pl.ds(sid * 640, 640)])

    @pl.when(sid == NS - 1)
    def _():
        pltpu.sync_copy(acc_sh.at[pl.ds((NS - 1) * 640, N - (NS - 1) * 640)],
                        out_hbm.at[cid, pl.ds((NS - 1) * 640,
                                              N - (NS - 1) * 640)])


_NB = 2000

def _combine_body(p_ref, o_ref):
    o_ref[...] = p_ref[0] + p_ref[1]


def _combine_call(parts):
    return pl.pallas_call(
        _combine_body,
        grid=(N // _NB,),
        in_specs=[pl.BlockSpec((NC, _NB, H), lambda i: (0, i, 0))],
        out_specs=pl.BlockSpec((_NB, H), lambda i: (i, 0)),
        out_shape=jax.ShapeDtypeStruct((N, H), jnp.float32),
    )(parts)


@jax.jit
def kernel(x_send, x_rec, index, edge_attr, gamma, beta, W1, b1, W2, b2):
    idx_flat = index.reshape(2 * E)
    counts2 = _hist_kernel(idx_flat, jnp.zeros((2 * N,), jnp.float32))
    counts = counts2.reshape(NC, 2, N)

    ea_r = edge_attr.reshape(E * NI // H, H)
    p_send, p_rec, se, b1e = _stats_call(
        counts, x_send, x_rec, ea_r, gamma.reshape(1, D), beta.reshape(1, D),
        W1, b1.reshape(1, H))

    pe = _pe_call(edge_attr, se, W1[2 * H:, :], b1e)

    b2v = jnp.broadcast_to(b2.reshape(1), (L,)).astype(jnp.float32)
    parts = _main_kernel(p_send, p_rec, pe, index[0], index[1], W2[:, 0], b2v,
                         jnp.zeros((N, H), jnp.float32))
    return _combine_call(parts)


# speedup vs baseline: 3.7765x; 1.2947x over previous
import functools

import jax
import jax.numpy as jnp
from jax import lax
from jax.experimental import pallas as pl
from jax.experimental.pallas import tpu as pltpu
from jax.experimental.pallas import tpu_sc as plsc

N = 10000
E = 320000
H = 128
NI = 16
D = 2 * H + NI

NC = 2
NS = 16
NW = NC * NS
L = 16

EPW = E // NW
CH = 25
NCHUNK = EPW // CH
NPH = 8
CPP = NCHUNK // NPH
DBLK = 16

FPW = E // NS
HCH = 125
NHCHUNK = FPW // HCH

_mesh = plsc.VectorSubcoreMesh(core_axis_name="c", subcore_axis_name="s")
_sc_params = pltpu.CompilerParams(needs_layout_passes=False)


@functools.partial(
    pl.kernel,
    out_type=jax.ShapeDtypeStruct((NC, N), jnp.float32),
    mesh=_mesh,
    scratch_types=[
        pltpu.VMEM((NHCHUNK, HCH), jnp.int32),
        pltpu.VMEM((HCH,), jnp.float32),
        pltpu.VMEM_SHARED((N,), jnp.float32),
        pltpu.SemaphoreType.DMA,
    ],
    compiler_params=_sc_params,
)
def _hist_kernel(idx_hbm, zeros_hbm, out_hbm, idx_v, ones_v, acc_sh, sem):
    cid = lax.axis_index("c")
    sid = lax.axis_index("s")

    @pl.when(sid == 0)
    def _():
        pltpu.sync_copy(zeros_hbm, acc_sh)

    for k in range(HCH // L):
        ones_v[pl.ds(k * L, L)] = jnp.full((L,), 1.0, jnp.float32)
    ones_v[pl.ds(HCH - L, L)] = jnp.full((L,), 1.0, jnp.float32)
    pltpu.sync_copy(idx_hbm.at[cid, sid], idx_v)
    plsc.subcore_barrier()

    def issue(i, carry):
        pltpu.async_copy(ones_v, acc_sh.at[idx_v.at[i]], sem, add=True)
        return carry

    lax.fori_loop(0, NHCHUNK, issue, 0)

    def drain(i, carry):
        pltpu.make_async_copy(ones_v, acc_sh.at[idx_v.at[0]], sem).wait()
        return carry

    lax.fori_loop(0, NHCHUNK, drain, 0)
    plsc.subcore_barrier()

    @pl.when(sid == 0)
    def _():
        pltpu.sync_copy(acc_sh, out_hbm.at[cid])


def _estats_body(ear_ref, ge_ref, be_ref, se_ref, she_ref):
    ear = ear_ref[...]
    se_cols = jnp.sum(ear, axis=0, keepdims=True)
    sq_cols = jnp.sum(ear * ear, axis=0, keepdims=True)
    sum_e = se_cols[:, 0:NI]
    sumsq_e = sq_cols[:, 0:NI]
    for g in range(1, H // NI):
        sum_e = sum_e + se_cols[:, g * NI:(g + 1) * NI]
        sumsq_e = sumsq_e + sq_cols[:, g * NI:(g + 1) * NI]
    mean_e = sum_e * (1.0 / E)
    var_e = sumsq_e * (1.0 / E) - mean_e * mean_e
    scale_e = ge_ref[...] * jax.lax.rsqrt(var_e + 1e-5)
    se_ref[...] = scale_e
    she_ref[...] = be_ref[...] - mean_e * scale_e


def _estats_call(ea_r, gamma_e, beta_e):
    return pl.pallas_call(
        _estats_body,
        out_shape=[
            jax.ShapeDtypeStruct((1, NI), jnp.float32),
            jax.ShapeDtypeStruct((1, NI), jnp.float32),
        ],
    )(ea_r, gamma_e, beta_e)


_EB = 8000

def _pe_body(ea_ref, se_ref, w1e_ref, pe_ref):
    pe_ref[...] = jnp.dot(ea_ref[...] * se_ref[...], w1e_ref[...],
                          preferred_element_type=jnp.float32)


def _pe_call(edge_attr, se, w1e):
    return pl.pallas_call(
        _pe_body,
        grid=(E // _EB,),
        in_specs=[
            pl.BlockSpec((_EB, NI), lambda i: (i, 0)),
            pl.BlockSpec((1, NI), lambda i: (0, 0)),
            pl.BlockSpec((NI, H), lambda i: (0, 0)),
        ],
        out_specs=pl.BlockSpec((_EB, H), lambda i: (i, 0)),
        out_shape=jax.ShapeDtypeStruct((E, H), jnp.float32),
    )(edge_attr, se, w1e)


def _stats_body(cnt_ref, xs_ref, xr_ref, g_ref, b_ref, w1_ref, b1_ref,
                she_ref, ps_ref, pr_ref):
    cs = cnt_ref[0:1, :]
    cr = cnt_ref[1:2, :]
    xs = xs_ref[...]
    xr = xr_ref[...]
    inv_e = 1.0 / E

    sum_s = jnp.dot(cs, xs, preferred_element_type=jnp.float32)
    sumsq_s = jnp.dot(cs, xs * xs, preferred_element_type=jnp.float32)
    sum_r = jnp.dot(cr, xr, preferred_element_type=jnp.float32)
    sumsq_r = jnp.dot(cr, xr * xr, preferred_element_type=jnp.float32)
    mean_s = sum_s * inv_e
    var_s = sumsq_s * inv_e - mean_s * mean_s
    mean_r = sum_r * inv_e
    var_r = sumsq_r * inv_e - mean_r * mean_r

    scale_s = g_ref[:, :H] * jax.lax.rsqrt(var_s + 1e-5)
    scale_r = g_ref[:, H:2 * H] * jax.lax.rsqrt(var_r + 1e-5)
    shift_s = b_ref[:, :H] - mean_s * scale_s
    shift_r = b_ref[:, H:2 * H] - mean_r * scale_r
    shift = jnp.concatenate([shift_s, shift_r, she_ref[...]], axis=1)

    b1e = b1_ref[...] + jnp.dot(shift, w1_ref[...],
                                preferred_element_type=jnp.float32)
    ps_ref[...] = jnp.dot(xs * scale_s, w1_ref[:H, :],
                          preferred_element_type=jnp.float32) + b1e
    pr_ref[...] = jnp.dot(xr * scale_r, w1_ref[H:2 * H, :],
                          preferred_element_type=jnp.float32)


def _stats_call(counts, x_send, x_rec, gamma2, beta2, W1, b12, she):
    return pl.pallas_call(
        _stats_body,
        out_shape=[
            jax.ShapeDtypeStruct((N, H), jnp.float32),
            jax.ShapeDtypeStruct((N, H), jnp.float32),
        ],
    )(counts, x_send, x_rec, gamma2, beta2, W1, b12, she)


@functools.partial(
    pl.kernel,
    out_type=jax.ShapeDtypeStruct((NC, N, H), jnp.float32),
    mesh=_mesh,
    scratch_types=[
        pltpu.VMEM((CPP, CH), jnp.int32),
        pltpu.VMEM((CPP, CH), jnp.int32),
        pltpu.VMEM((CH, H), jnp.float32),
        pltpu.VMEM((CH, H), jnp.float32),
        pltpu.VMEM((CH, H), jnp.float32),
        pltpu.VMEM((CH, H), jnp.float32),
        pltpu.VMEM((CH, H), jnp.float32),
        pltpu.VMEM((CH, H), jnp.float32),
        pltpu.VMEM((CH, H), jnp.float32),
        pltpu.VMEM((CH, H), jnp.float32),
        pltpu.VMEM((DBLK, H), jnp.float32),
        pltpu.VMEM((H,), jnp.float32),
        pltpu.VMEM((L,), jnp.float32),
        pltpu.VMEM_SHARED((N, H), jnp.float32),
        pltpu.SemaphoreType.DMA,
        pltpu.SemaphoreType.DMA,
        pltpu.SemaphoreType.DMA,
        pltpu.SemaphoreType.DMA,
        pltpu.SemaphoreType.DMA,
        pltpu.SemaphoreType.DMA,
        pltpu.SemaphoreType.DMA,
        pltpu.SemaphoreType.DMA,
    ],
    compiler_params=_sc_params,
)
def _main_kernel(ps_hbm, pr_hbm, pe_hbm, is_hbm, ir_hbm, w2_hbm, b2_hbm,
                 zrow_hbm, out_hbm, is_v, ir_v, rs0, rs1, rr0, rr1, rpe0,
                 rpe1, ro0, ro1, dr_v, w2_v, b2_v, acc_sh,
                 sg00, sg01, sg02, sg10, sg11, sg12, ssc0, ssc1):
    cid = lax.axis_index("c")
    sid = lax.axis_index("s")
    wid = cid * NS + sid

    rs = (rs0, rs1)
    rr = (rr0, rr1)
    rpe = (rpe0, rpe1)
    ro = (ro0, ro1)
    sg = ((sg00, sg01, sg02), (sg10, sg11, sg12))
    ssc = (ssc0, ssc1)

    pltpu.sync_copy(zrow_hbm, ro0)
    zbase = sid * (N // NS)

    def zero_body(r, carry):
        pltpu.sync_copy(ro0, acc_sh.at[pl.ds(zbase + r * CH, CH)])
        return carry

    lax.fori_loop(0, (N // NS) // CH, zero_body, 0)
    if (N // NS) % CH:
        pltpu.sync_copy(ro0.at[pl.ds(0, (N // NS) % CH)],
                        acc_sh.at[pl.ds(zbase + ((N // NS) // CH) * CH,
                                        (N // NS) % CH)])

    pltpu.sync_copy(w2_hbm, w2_v)
    pltpu.sync_copy(b2_hbm, b2_v)
    plsc.subcore_barrier()

    b2v = b2_v[pl.ds(0, L)]

    def g_descs(p, ph, c):
        cg = (wid * NPH + ph) * CPP + c
        return (
            pltpu.make_async_copy(ps_hbm.at[is_v.at[c]], rs[p], sg[p][0]),
            pltpu.make_async_copy(pr_hbm.at[ir_v.at[c]], rr[p], sg[p][1]),
            pltpu.make_async_copy(pe_hbm.at[cg], rpe[p], sg[p][2]),
        )

    def issue_gather(p, ph, c):
        for d in g_descs(p, ph, c):
            d.start()

    def wait_gather(p):
        for d in g_descs(p, 0, 0):
            d.wait()

    def issue_scatter(p, c):
        pltpu.async_copy(ro[p], acc_sh.at[ir_v.at[c]], ssc[p], add=True)

    def wait_scatter(p):
        pltpu.make_async_copy(ro[p], acc_sh.at[ir_v.at[0]], ssc[p]).wait()

    def compute(p):
        rs_p, rr_p, rpe_p, ro_p = rs[p], rr[p], rpe[p], ro[p]

        @plsc.parallel_loop(0, CH)
        def _(e):
            acc = jnp.zeros((L,), jnp.float32)
            ms = []
            for j in range(H // L):
                sl = pl.ds(j * L, L)
                z = rs_p[e, sl] + rr_p[e, sl] + rpe_p[e, sl]
                m = z / (1.0 + jnp.exp(-z))
                ms.append(m)
                acc = acc + m * w2_v[sl]
            tv = jax.lax.broadcast(jnp.sum(acc), (L,)) + b2v
            w = 1.0 / (1.0 + jnp.exp(-tv))
            for j in range(H // L):
                ro_p[e, pl.ds(j * L, L)] = ms[j] * w

    def phase_body(ph, carry):
        @pl.when(ph > 0)
        def _():
            wait_scatter(0)
            wait_scatter(1)

        pltpu.sync_copy(is_hbm.at[wid, ph], is_v)
        pltpu.sync_copy(ir_hbm.at[wid, ph], ir_v)
        issue_gather(0, ph, 0)
        issue_gather(1, ph, 1)

        def pair_body(k, carry2):
            for p in (0, 1):
                c = 2 * k + p
                wait_gather(p)

                @pl.when(k > 0)
                def _():
                    wait_scatter(p)

                compute(p)
                issue_scatter(p, c)

                @pl.when(c + 2 < CPP)
                def _():
                    issue_gather(p, ph, c + 2)
            return carry2

        lax.fori_loop(0, CPP // 2, pair_body, 0)
        return carry

    lax.fori_loop(0, NPH, phase_body, 0)
    wait_scatter(0)
    wait_scatter(1)
    plsc.subcore_barrier()

    dbase = sid * 640

    def drain_body(r, carry):
        off = dbase + r * DBLK
        pltpu.sync_copy(acc_sh.at[pl.ds(off, DBLK)], dr_v)
        pltpu.sync_copy(dr_v, out_hbm.at[cid, pl.ds(off, DBLK)])
        return carry

    nblk = jnp.where(sid == NS - 1, (N - (NS - 1) * 640) // DBLK, 640 // DBLK)
    lax.fori_loop(0, nblk, drain_body, 0)


_NB = 2000

def _combine_body(p_ref, o_ref):
    o_ref[...] = p_ref[0] + p_ref[1]


def _combine_call(parts):
    return pl.pallas_call(
        _combine_body,
        grid=(N // _NB,),
        in_specs=[pl.BlockSpec((NC, _NB, H), lambda i: (0, i, 0))],
        out_specs=pl.BlockSpec((_NB, H), lambda i: (i, 0)),
        out_shape=jax.ShapeDtypeStruct((N, H), jnp.float32),
    )(parts)


@jax.jit
def kernel(x_send, x_rec, index, edge_attr, gamma, beta, W1, b1, W2, b2):
    gamma2 = gamma.reshape(1, D)
    beta2 = beta.reshape(1, D)
    ea_r = edge_attr.reshape(E * NI // H, H)

    se, she = _estats_call(ea_r, gamma2[:, 2 * H:], beta2[:, 2 * H:])
    pe = _pe_call(edge_attr, se, W1[2 * H:, :])

    idx3 = index.reshape(NC, NS, NHCHUNK, HCH)
    counts = _hist_kernel(idx3, jnp.zeros((N,), jnp.float32))

    p_send, p_rec = _stats_call(counts, x_send, x_rec, gamma2, beta2, W1,
                                b1.reshape(1, H), she)

    b2v = jnp.broadcast_to(b2.reshape(1), (L,)).astype(jnp.float32)
    parts = _main_kernel(
        p_send, p_rec, pe.reshape(E // CH, CH, H),
        index[0].reshape(NW, NPH, CPP, CH), index[1].reshape(NW, NPH, CPP, CH),
        W2[:, 0], b2v, jnp.zeros((CH, H), jnp.float32))
    return _combine_call(parts)


# speedup vs baseline: 4.9574x; 1.3127x over previous
import functools

import jax
import jax.numpy as jnp
from jax import lax
from jax.experimental import pallas as pl
from jax.experimental.pallas import tpu as pltpu
from jax.experimental.pallas import tpu_sc as plsc

N = 10000
E = 320000
H = 128
NI = 16
D = 2 * H + NI

NC = 2
NS = 16
NW = NC * NS
L = 16

EPW = E // NW
CH = 25
NCHUNK = EPW // CH
NPH = 8
CPP = NCHUNK // NPH
DBLK = 16

FPW = E // NS
HCH = 125
NHCHUNK = FPW // HCH

_mesh = plsc.VectorSubcoreMesh(core_axis_name="c", subcore_axis_name="s")
_sc_params = pltpu.CompilerParams(needs_layout_passes=False)


@functools.partial(
    pl.kernel,
    out_type=jax.ShapeDtypeStruct((NC, N), jnp.float32),
    mesh=_mesh,
    scratch_types=[
        pltpu.VMEM((NHCHUNK, HCH), jnp.int32),
        pltpu.VMEM((HCH,), jnp.float32),
        pltpu.VMEM_SHARED((N,), jnp.float32),
        pltpu.SemaphoreType.DMA,
    ],
    compiler_params=_sc_params,
)
def _hist_kernel(idx_hbm, zeros_hbm, out_hbm, idx_v, ones_v, acc_sh, sem):
    cid = lax.axis_index("c")
    sid = lax.axis_index("s")

    @pl.when(sid == 0)
    def _():
        pltpu.sync_copy(zeros_hbm, acc_sh)

    for k in range(HCH // L):
        ones_v[pl.ds(k * L, L)] = jnp.full((L,), 1.0, jnp.float32)
    ones_v[pl.ds(HCH - L, L)] = jnp.full((L,), 1.0, jnp.float32)
    pltpu.sync_copy(idx_hbm.at[cid, sid], idx_v)
    plsc.subcore_barrier()

    def issue(i, carry):
        pltpu.async_copy(ones_v, acc_sh.at[idx_v.at[i]], sem, add=True)
        return carry

    lax.fori_loop(0, NHCHUNK, issue, 0)

    def drain(i, carry):
        pltpu.make_async_copy(ones_v, acc_sh.at[idx_v.at[0]], sem).wait()
        return carry

    lax.fori_loop(0, NHCHUNK, drain, 0)
    plsc.subcore_barrier()

    @pl.when(sid == 0)
    def _():
        pltpu.sync_copy(acc_sh, out_hbm.at[cid])


_EB = 8000
_NEB = E // _EB

def _edge_body(ea_ref, ge_ref, be_ref, w1e_ref, pe_ref, she_ref,
               acc_ref, w1s_ref):
    k = pl.program_id(0)

    @pl.when(k == 0)
    def _():
        acc_ref[...] = jnp.zeros_like(acc_ref)

    ea = ea_ref[...]

    @pl.when(k < _NEB)
    def _():
        acc_ref[0:1, :] += jnp.sum(ea, axis=0, keepdims=True)
        acc_ref[1:2, :] += jnp.sum(ea * ea, axis=0, keepdims=True)

    @pl.when(k == _NEB)
    def _():
        inv_e = 1.0 / E
        mean_e = acc_ref[0:1, :] * inv_e
        var_e = acc_ref[1:2, :] * inv_e - mean_e * mean_e
        scale_e = ge_ref[...] * jax.lax.rsqrt(var_e + 1e-5)
        she_ref[...] = be_ref[...] - mean_e * scale_e
        ii = jax.lax.broadcasted_iota(jnp.int32, (NI, NI), 0)
        jj = jax.lax.broadcasted_iota(jnp.int32, (NI, NI), 1)
        diag = jnp.where(ii == jj, 1.0, 0.0) * scale_e
        w1s_ref[...] = jnp.dot(diag, w1e_ref[...],
                               preferred_element_type=jnp.float32)

    @pl.when(k >= _NEB)
    def _():
        pe_ref[...] = jnp.dot(ea, w1s_ref[...],
                              preferred_element_type=jnp.float32)


def _edge_call(edge_attr, gamma_e, beta_e, w1e):
    return pl.pallas_call(
        _edge_body,
        grid=(2 * _NEB,),
        in_specs=[
            pl.BlockSpec((_EB, NI),
                         lambda k: (jnp.where(k < _NEB, k, k - _NEB), 0)),
            pl.BlockSpec((1, NI), lambda k: (0, 0)),
            pl.BlockSpec((1, NI), lambda k: (0, 0)),
            pl.BlockSpec((NI, H), lambda k: (0, 0)),
        ],
        out_specs=[
            pl.BlockSpec((_EB, H),
                         lambda k: (jnp.where(k < _NEB, 0, k - _NEB), 0)),
            pl.BlockSpec((1, NI), lambda k: (0, 0)),
        ],
        out_shape=[
            jax.ShapeDtypeStruct((E, H), jnp.float32),
            jax.ShapeDtypeStruct((1, NI), jnp.float32),
        ],
        scratch_shapes=[
            pltpu.VMEM((8, NI), jnp.float32),
            pltpu.VMEM((NI, H), jnp.float32),
        ],
    )(edge_attr, gamma_e, beta_e, w1e)


def _stats_body(cnt_ref, xs_ref, xr_ref, g_ref, b_ref, w1_ref, b1_ref,
                she_ref, ps_ref, pr_ref):
    cs = cnt_ref[0:1, :]
    cr = cnt_ref[1:2, :]
    xs = xs_ref[...]
    xr = xr_ref[...]
    inv_e = 1.0 / E

    sum_s = jnp.dot(cs, xs, preferred_element_type=jnp.float32)
    sumsq_s = jnp.dot(cs, xs * xs, preferred_element_type=jnp.float32)
    sum_r = jnp.dot(cr, xr, preferred_element_type=jnp.float32)
    sumsq_r = jnp.dot(cr, xr * xr, preferred_element_type=jnp.float32)
    mean_s = sum_s * inv_e
    var_s = sumsq_s * inv_e - mean_s * mean_s
    mean_r = sum_r * inv_e
    var_r = sumsq_r * inv_e - mean_r * mean_r

    scale_s = g_ref[:, :H] * jax.lax.rsqrt(var_s + 1e-5)
    scale_r = g_ref[:, H:2 * H] * jax.lax.rsqrt(var_r + 1e-5)
    shift_s = b_ref[:, :H] - mean_s * scale_s
    shift_r = b_ref[:, H:2 * H] - mean_r * scale_r
    shift = jnp.concatenate([shift_s, shift_r, she_ref[...]], axis=1)

    b1e = b1_ref[...] + jnp.dot(shift, w1_ref[...],
                                preferred_element_type=jnp.float32)
    ps_ref[...] = jnp.dot(xs * scale_s, w1_ref[:H, :],
                          preferred_element_type=jnp.float32) + b1e
    pr_ref[...] = jnp.dot(xr * scale_r, w1_ref[H:2 * H, :],
                          preferred_element_type=jnp.float32)


def _stats_call(counts, x_send, x_rec, gamma2, beta2, W1, b12, she):
    return pl.pallas_call(
        _stats_body,
        out_shape=[
            jax.ShapeDtypeStruct((N, H), jnp.float32),
            jax.ShapeDtypeStruct((N, H), jnp.float32),
        ],
    )(counts, x_send, x_rec, gamma2, beta2, W1, b12, she)


@functools.partial(
    pl.kernel,
    out_type=jax.ShapeDtypeStruct((NC, N, H), jnp.float32),
    mesh=_mesh,
    scratch_types=[
        pltpu.VMEM((CPP, CH), jnp.int32),
        pltpu.VMEM((CPP, CH), jnp.int32),
        pltpu.VMEM((CH, H), jnp.float32),
        pltpu.VMEM((CH, H), jnp.float32),
        pltpu.VMEM((CH, H), jnp.float32),
        pltpu.VMEM((CH, H), jnp.float32),
        pltpu.VMEM((CH, H), jnp.float32),
        pltpu.VMEM((CH, H), jnp.float32),
        pltpu.VMEM((CH, H), jnp.float32),
        pltpu.VMEM((CH, H), jnp.float32),
        pltpu.VMEM((2 * L,), jnp.int32),
        pltpu.VMEM((2 * L,), jnp.int32),
        pltpu.VMEM((DBLK, H), jnp.float32),
        pltpu.VMEM((H,), jnp.float32),
        pltpu.VMEM((L,), jnp.float32),
        pltpu.VMEM_SHARED((N, H), jnp.float32),
        pltpu.SemaphoreType.DMA,
        pltpu.SemaphoreType.DMA,
        pltpu.SemaphoreType.DMA,
        pltpu.SemaphoreType.DMA,
        pltpu.SemaphoreType.DMA,
        pltpu.SemaphoreType.DMA,
        pltpu.SemaphoreType.DMA,
        pltpu.SemaphoreType.DMA,
    ],
    compiler_params=_sc_params,
)
def _main_kernel(ps_hbm, pr_hbm, pe_hbm, is_hbm, ir_hbm, w2_hbm, b2_hbm,
                 zrow_hbm, out_hbm, is_v, ir_v, rs0, rs1, rr0, rr1, rpe0,
                 rpe1, ro0, ro1, pi0, pi1, dr_v, w2_v, b2_v, acc_sh,
                 sg00, sg01, sg02, sg10, sg11, sg12, ssc0, ssc1):
    cid = lax.axis_index("c")
    sid = lax.axis_index("s")
    wid = cid * NS + sid

    rs = (rs0, rs1)
    rr = (rr0, rr1)
    rpe = (rpe0, rpe1)
    ro = (ro0, ro1)
    pi = (pi0, pi1)
    sg = ((sg00, sg01, sg02), (sg10, sg11, sg12))
    ssc = (ssc0, ssc1)

    pltpu.sync_copy(zrow_hbm, ro0)
    zbase = sid * (N // NS)

    def zero_body(r, carry):
        pltpu.sync_copy(ro0, acc_sh.at[pl.ds(zbase + r * CH, CH)])
        return carry

    lax.fori_loop(0, (N // NS) // CH, zero_body, 0)
    if (N // NS) % CH:
        pltpu.sync_copy(ro0.at[pl.ds(0, (N // NS) % CH)],
                        acc_sh.at[pl.ds(zbase + ((N // NS) // CH) * CH,
                                        (N // NS) % CH)])

    pltpu.sync_copy(w2_hbm, w2_v)
    pltpu.sync_copy(b2_hbm, b2_v)
    plsc.subcore_barrier()

    b2v = b2_v[pl.ds(0, L)]

    def g_descs(p, ph, c):
        return (
            pltpu.make_async_copy(ps_hbm.at[is_v.at[c]], rs[p], sg[p][0]),
            pltpu.make_async_copy(pr_hbm.at[ir_v.at[c]], rr[p], sg[p][1]),
            pltpu.make_async_copy(pe_hbm.at[pi[p].at[pl.ds(0, CH)]], rpe[p],
                                  sg[p][2]),
        )

    def issue_gather(p, ph, c):
        eb = (wid * NPH + ph) * CPP * CH + c * CH
        iota = jax.lax.iota(jnp.int32, L)
        pi[p][pl.ds(0, L)] = iota + eb
        pi[p][pl.ds(L, L)] = iota + (eb + L)
        for d in g_descs(p, ph, c):
            d.start()

    def wait_gather(p):
        for d in g_descs(p, 0, 0):
            d.wait()

    def issue_scatter(p, c):
        pltpu.async_copy(ro[p], acc_sh.at[ir_v.at[c]], ssc[p], add=True)

    def wait_scatter(p):
        pltpu.make_async_copy(ro[p], acc_sh.at[ir_v.at[0]], ssc[p]).wait()

    def compute(p):
        rs_p, rr_p, rpe_p, ro_p = rs[p], rr[p], rpe[p], ro[p]

        @plsc.parallel_loop(0, CH)
        def _(e):
            acc = jnp.zeros((L,), jnp.float32)
            ms = []
            for j in range(H // L):
                sl = pl.ds(j * L, L)
                z = rs_p[e, sl] + rr_p[e, sl] + rpe_p[e, sl]
                m = z / (1.0 + jnp.exp(-z))
                ms.append(m)
                acc = acc + m * w2_v[sl]
            tv = jax.lax.broadcast(jnp.sum(acc), (L,)) + b2v
            w = 1.0 / (1.0 + jnp.exp(-tv))
            for j in range(H // L):
                ro_p[e, pl.ds(j * L, L)] = ms[j] * w

    def phase_body(ph, carry):
        @pl.when(ph > 0)
        def _():
            wait_scatter(0)
            wait_scatter(1)

        pltpu.sync_copy(is_hbm.at[wid, ph], is_v)
        pltpu.sync_copy(ir_hbm.at[wid, ph], ir_v)
        issue_gather(0, ph, 0)
        issue_gather(1, ph, 1)

        def pair_body(k, carry2):
            for p in (0, 1):
                c = 2 * k + p
                wait_gather(p)

                @pl.when(k > 0)
                def _():
                    wait_scatter(p)

                compute(p)
                issue_scatter(p, c)

                @pl.when(c + 2 < CPP)
                def _():
                    issue_gather(p, ph, c + 2)
            return carry2

        lax.fori_loop(0, CPP // 2, pair_body, 0)
        return carry

    lax.fori_loop(0, NPH, phase_body, 0)
    wait_scatter(0)
    wait_scatter(1)
    plsc.subcore_barrier()

    dbase = sid * 640

    def drain_body(r, carry):
        off = dbase + r * DBLK
        pltpu.sync_copy(acc_sh.at[pl.ds(off, DBLK)], dr_v)
        pltpu.sync_copy(dr_v, out_hbm.at[cid, pl.ds(off, DBLK)])
        return carry

    nblk = jnp.where(sid == NS - 1, (N - (NS - 1) * 640) // DBLK, 640 // DBLK)
    lax.fori_loop(0, nblk, drain_body, 0)


_NB = 2000

def _combine_body(p_ref, o_ref):
    o_ref[...] = p_ref[0] + p_ref[1]


def _combine_call(parts):
    return pl.pallas_call(
        _combine_body,
        grid=(N // _NB,),
        in_specs=[pl.BlockSpec((NC, _NB, H), lambda i: (0, i, 0))],
        out_specs=pl.BlockSpec((_NB, H), lambda i: (i, 0)),
        out_shape=jax.ShapeDtypeStruct((N, H), jnp.float32),
    )(parts)


@jax.jit
def kernel(x_send, x_rec, index, edge_attr, gamma, beta, W1, b1, W2, b2):
    gamma2 = gamma.reshape(1, D)
    beta2 = beta.reshape(1, D)

    pe, she = _edge_call(edge_attr, gamma2[:, 2 * H:], beta2[:, 2 * H:],
                         W1[2 * H:, :])

    idx3 = index.reshape(NC, NS, NHCHUNK, HCH)
    counts = _hist_kernel(idx3, jnp.zeros((N,), jnp.float32))

    p_send, p_rec = _stats_call(counts, x_send, x_rec, gamma2, beta2, W1,
                                b1.reshape(1, H), she)

    b2v = jnp.broadcast_to(b2.reshape(1), (L,)).astype(jnp.float32)
    parts = _main_kernel(
        p_send, p_rec, pe,
        index[0].reshape(NW, NPH, CPP, CH), index[1].reshape(NW, NPH, CPP, CH),
        W2[:, 0], b2v, jnp.zeros((CH, H), jnp.float32))
    return _combine_call(parts)
